# baseline (speedup 1.0000x reference)
import functools

import jax
import jax.numpy as jnp
from jax import lax
from jax.experimental import pallas as pl
from jax.experimental.pallas import tpu as pltpu
from jax.experimental.pallas import tpu_sc as plsc

_VOCAB = 49408
_D = 768
_MAXLEN = 77
_B = 1024
_L = 77
_N = _B * _L
_NW = 32
_PER_W = _N // _NW
_C = 32
_NCH = _PER_W // _C
_LANES = 16


def _body(tok_ids, pos_ids, tok_tab, pos_tab, out, tidx, pidx, tbuf, pbuf,
          sem_t, sem_p):
  wid = lax.axis_index("s") * 2 + lax.axis_index("c")
  base = wid * _PER_W

  pltpu.sync_copy(tok_ids.at[wid], tidx)
  pltpu.sync_copy(pos_ids.at[wid], pidx)

  @pl.loop(0, _NCH)
  def _chunk(g):
    ct = pltpu.async_copy(tok_tab.at[tidx.at[g]], tbuf, sem_t)
    cp = pltpu.async_copy(pos_tab.at[pidx.at[g]], pbuf, sem_p)
    ct.wait()
    cp.wait()

    @pl.loop(0, _C)
    def _row(r):
      @pl.loop(0, _D // _LANES, unroll=8)
      def _col(k):
        off = pl.multiple_of(k * _LANES, _LANES)
        tbuf[r, pl.ds(off, _LANES)] = (
            tbuf[r, pl.ds(off, _LANES)] + pbuf[r, pl.ds(off, _LANES)])

    pltpu.sync_copy(tbuf, out.at[pl.ds(base + g * _C, _C)])


@jax.jit
def kernel(input_ids, position_ids, token_table, position_table):
  tok = input_ids.reshape(_NW, _NCH, _C).astype(jnp.int32)
  pos = position_ids.reshape(_NW, _NCH, _C).astype(jnp.int32)

  mesh = plsc.VectorSubcoreMesh(core_axis_name="c", subcore_axis_name="s")
  kern = functools.partial(
      pl.kernel,
      out_type=jax.ShapeDtypeStruct((_N, _D), jnp.float32),
      mesh=mesh,
      scratch_types=[
          pltpu.VMEM((_NCH, _C), jnp.int32),
          pltpu.VMEM((_NCH, _C), jnp.int32),
          pltpu.VMEM((_C, _D), jnp.float32),
          pltpu.VMEM((_C, _D), jnp.float32),
          pltpu.SemaphoreType.DMA,
          pltpu.SemaphoreType.DMA,
      ],
  )(_body)
  flat = kern(tok, pos, token_table, position_table)
  return flat.reshape(_B, _L, _D)


# speedup vs baseline: 1.0309x; 1.0309x over previous
import functools

import jax
import jax.numpy as jnp
from jax import lax
from jax.experimental import pallas as pl
from jax.experimental.pallas import tpu as pltpu
from jax.experimental.pallas import tpu_sc as plsc

_VOCAB = 49408
_D = 768
_MAXLEN = 77
_B = 1024
_L = 77
_N = _B * _L
_NW = 32
_PER_W = _N // _NW
_C = 16
_NCH = _PER_W // _C
_LANES = 16


def _body(tok_ids, pos_ids, tok_tab, pos_tab, out, tidx, pidx,
          tb0, tb1, pb0, pb1, st0, st1, sp0, sp1, so0, so1):
  wid = lax.axis_index("s") * 2 + lax.axis_index("c")
  base = wid * _PER_W

  pltpu.sync_copy(tok_ids.at[wid], tidx)
  pltpu.sync_copy(pos_ids.at[wid], pidx)

  slots = ((tb0, pb0, st0, sp0, so0), (tb1, pb1, st1, sp1, so1))

  def gstart(g, tb, pb, st, sp):
    pltpu.async_copy(tok_tab.at[tidx.at[g]], tb, st)
    pltpu.async_copy(pos_tab.at[pidx.at[g]], pb, sp)

  def gwait(g, tb, pb, st, sp):
    pltpu.make_async_copy(tok_tab.at[tidx.at[g]], tb, st).wait()
    pltpu.make_async_copy(pos_tab.at[pidx.at[g]], pb, sp).wait()

  def sstart(g, tb, so):
    pltpu.async_copy(tb, out.at[pl.ds(base + g * _C, _C)], so)

  def swait(tb, so):
    pltpu.make_async_copy(tb, out.at[pl.ds(base, _C)], so).wait()

  gstart(0, tb0, pb0, st0, sp0)

  @pl.loop(0, _NCH, step=2)
  def _pair(g):
    for k in range(2):
      gk = g + k
      tb, pb, st, sp, so = slots[k]
      tb2, pb2, st2, sp2, so2 = slots[1 - k]

      gwait(gk, tb, pb, st, sp)

      @pl.loop(0, _C)
      def _row(r):
        @pl.loop(0, _D // _LANES, unroll=8)
        def _col(kk):
          off = pl.multiple_of(kk * _LANES, _LANES)
          tb[r, pl.ds(off, _LANES)] = (
              tb[r, pl.ds(off, _LANES)] + pb[r, pl.ds(off, _LANES)])

      @pl.when(gk + 1 < _NCH)
      def _prefetch():
        @pl.when(gk >= 1)
        def _drain_prev_store():
          swait(tb2, so2)
        gstart(gk + 1, tb2, pb2, st2, sp2)

      sstart(gk, tb, so)

  swait(tb0, so0)
  swait(tb1, so1)


@jax.jit
def kernel(input_ids, position_ids, token_table, position_table):
  tok = input_ids.reshape(_NW, _NCH, _C).astype(jnp.int32)
  pos = position_ids.reshape(_NW, _NCH, _C).astype(jnp.int32)

  mesh = plsc.VectorSubcoreMesh(core_axis_name="c", subcore_axis_name="s")
  kern = functools.partial(
      pl.kernel,
      out_type=jax.ShapeDtypeStruct((_N, _D), jnp.float32),
      mesh=mesh,
      scratch_types=[
          pltpu.VMEM((_NCH, _C), jnp.int32),
          pltpu.VMEM((_NCH, _C), jnp.int32),
          pltpu.VMEM((_C, _D), jnp.float32),
          pltpu.VMEM((_C, _D), jnp.float32),
          pltpu.VMEM((_C, _D), jnp.float32),
          pltpu.VMEM((_C, _D), jnp.float32),
          pltpu.SemaphoreType.DMA,
          pltpu.SemaphoreType.DMA,
          pltpu.SemaphoreType.DMA,
          pltpu.SemaphoreType.DMA,
          pltpu.SemaphoreType.DMA,
          pltpu.SemaphoreType.DMA,
      ],
  )(_body)
  flat = kern(tok, pos, token_table, position_table)
  return flat.reshape(_B, _L, _D)
